# Initial kernel scaffold; baseline (speedup 1.0000x reference)
import jax
import jax.numpy as jnp
from jax.experimental import pallas as pl


def kernel(x, edge_index, batch, W0, b0, gamma0, beta0, W1, b1, gamma1, beta1, W2, b2, gamma2, beta2):
    raise NotImplementedError("write your pallas kernel here")


# speedup vs baseline: 10.7924x; 10.7924x over previous
import functools

import jax
import jax.numpy as jnp
from jax import lax
from jax.experimental import pallas as pl
from jax.experimental.pallas import tpu as pltpu
from jax.experimental.pallas import tpu_sc as plsc

N = 10000
E = 320000
D = 128
G = 64
EPSV = 1e-5

NC = 2
NS = 16
NW = NC * NS
E_W = E // NW
CH = 80
NCH = E_W // CH
S0 = 624
S_LAST = N - S0 * (NS - 1)

NB = 5
BR = N // NB

def _sc_degree_body(col_hbm, ones_hbm, zeros_hbm, out_hbm, col_v, ones_v, acc):
    c = lax.axis_index("c")
    s = lax.axis_index("s")
    @pl.when(s < NS - 1)
    def _():
        pltpu.sync_copy(zeros_hbm.at[pl.ds(s * S0, S0)], acc.at[pl.ds(s * S0, S0)])

    @pl.when(s == NS - 1)
    def _():
        pltpu.sync_copy(zeros_hbm.at[pl.ds((NS - 1) * S0, S_LAST)],
                        acc.at[pl.ds((NS - 1) * S0, S_LAST)])

    pltpu.sync_copy(ones_hbm, ones_v)
    plsc.subcore_barrier()
    wid = c * NS + s

    def body(i, carry):
        eb = wid * E_W + i * CH
        pltpu.sync_copy(col_hbm.at[pl.ds(eb, CH)], col_v)
        pltpu.sync_copy(ones_v, acc.at[col_v], add=True)
        return carry

    lax.fori_loop(0, NCH, body, 0)
    plsc.subcore_barrier()

    @pl.when(s < NS - 1)
    def _():
        pltpu.sync_copy(acc.at[pl.ds(s * S0, S0)], out_hbm.at[c, pl.ds(s * S0, S0)])

    @pl.when(s == NS - 1)
    def _():
        pltpu.sync_copy(acc.at[pl.ds((NS - 1) * S0, S_LAST)],
                        out_hbm.at[c, pl.ds((NS - 1) * S0, S_LAST)])


def _sc_agg_body(y_hbm, row_hbm, col_hbm, zeros_hbm, out_hbm, row_v, col_v, rows_v, acc, sem):
    c = lax.axis_index("c")
    s = lax.axis_index("s")
    @pl.when(s < NS - 1)
    def _():
        pltpu.sync_copy(zeros_hbm.at[pl.ds(s * S0, S0)], acc.at[pl.ds(s * S0, S0)])

    @pl.when(s == NS - 1)
    def _():
        pltpu.sync_copy(zeros_hbm.at[pl.ds((NS - 1) * S0, S_LAST)],
                        acc.at[pl.ds((NS - 1) * S0, S_LAST)])

    plsc.subcore_barrier()
    wid = c * NS + s

    def body(i, carry):
        eb = wid * E_W + i * CH
        pltpu.sync_copy(row_hbm.at[pl.ds(eb, CH)], row_v)
        pltpu.sync_copy(col_hbm.at[pl.ds(eb, CH)], col_v)
        pltpu.async_copy(y_hbm.at[row_v], rows_v, sem).wait()
        pltpu.sync_copy(rows_v, acc.at[col_v], add=True)
        return carry

    lax.fori_loop(0, NCH, body, 0)
    plsc.subcore_barrier()

    @pl.when(s < NS - 1)
    def _():
        pltpu.sync_copy(acc.at[pl.ds(s * S0, S0)], out_hbm.at[c, pl.ds(s * S0, S0)])

    @pl.when(s == NS - 1)
    def _():
        pltpu.sync_copy(acc.at[pl.ds((NS - 1) * S0, S_LAST)],
                        out_hbm.at[c, pl.ds((NS - 1) * S0, S_LAST)])


@functools.cache
def _sc_kernels():
    mesh = plsc.VectorSubcoreMesh(
        core_axis_name="c", subcore_axis_name="s", num_cores=NC, num_subcores=NS
    )
    sc_degree = pl.kernel(
        _sc_degree_body,
        out_type=jax.ShapeDtypeStruct((NC, N, 16), jnp.float32),
        mesh=mesh,
        compiler_params=pltpu.CompilerParams(use_tc_tiling_on_sc=False),
        scratch_types=[
            pltpu.VMEM((CH,), jnp.int32),
            pltpu.VMEM((CH, 16), jnp.float32),
            pltpu.VMEM_SHARED((N, 16), jnp.float32),
        ],
    )
    sc_agg = pl.kernel(
        _sc_agg_body,
        out_type=jax.ShapeDtypeStruct((NC, N, D), jnp.float32),
        mesh=mesh,
        scratch_types=[
            pltpu.VMEM((CH,), jnp.int32),
            pltpu.VMEM((CH,), jnp.int32),
            pltpu.VMEM((CH, D), jnp.float32),
            pltpu.VMEM_SHARED((N, D), jnp.float32),
            pltpu.SemaphoreType.DMA,
        ],
    )
    return sc_degree, sc_agg


def _tc0_body(degp_ref, x_ref, w_ref, y_ref, dinv_ref):
    dp = degp_ref[...]
    deg = dp[0, :, 0:1] + dp[1, :, 0:1] + 1.0
    dinv = lax.rsqrt(deg)
    dvb = jnp.broadcast_to(dinv, (BR, D))
    dinv_ref[...] = dvb
    y_ref[...] = dvb * jnp.dot(x_ref[...], w_ref[...], preferred_element_type=jnp.float32)


def _tc0(degp, x, w0):
    return pl.pallas_call(
        _tc0_body,
        grid=(NB,),
        in_specs=[
            pl.BlockSpec((NC, BR, 16), lambda i: (0, i, 0)),
            pl.BlockSpec((BR, D), lambda i: (i, 0)),
            pl.BlockSpec((D, D), lambda i: (0, 0)),
        ],
        out_specs=[
            pl.BlockSpec((BR, D), lambda i: (i, 0)),
            pl.BlockSpec((BR, D), lambda i: (i, 0)),
        ],
        out_shape=[
            jax.ShapeDtypeStruct((N, D), jnp.float32),
            jax.ShapeDtypeStruct((N, D), jnp.float32),
        ],
    )(degp, x, w0)


def _tca_body(a0_ref, a1_ref, y_ref, dinv_ref, b_ref, s_ref, stats_ref):
    i = pl.program_id(0)
    s = dinv_ref[...] * (a0_ref[...] + a1_ref[...] + y_ref[...]) + b_ref[...]
    s_ref[...] = s
    part = jnp.concatenate(
        [jnp.sum(s, axis=0, keepdims=True), jnp.sum(s * s, axis=0, keepdims=True)],
        axis=0,
    )

    @pl.when(i == 0)
    def _():
        stats_ref[...] = part

    @pl.when(i > 0)
    def _():
        stats_ref[...] = stats_ref[...] + part


def _tca(a0, a1, y, dinv, b):
    return pl.pallas_call(
        _tca_body,
        grid=(NB,),
        in_specs=[
            pl.BlockSpec((BR, D), lambda i: (i, 0)),
            pl.BlockSpec((BR, D), lambda i: (i, 0)),
            pl.BlockSpec((BR, D), lambda i: (i, 0)),
            pl.BlockSpec((BR, D), lambda i: (i, 0)),
            pl.BlockSpec((1, D), lambda i: (0, 0)),
        ],
        out_specs=[
            pl.BlockSpec((BR, D), lambda i: (i, 0)),
            pl.BlockSpec((2, D), lambda i: (0, 0)),
        ],
        out_shape=[
            jax.ShapeDtypeStruct((N, D), jnp.float32),
            jax.ShapeDtypeStruct((2, D), jnp.float32),
        ],
    )(a0, a1, y, dinv, b)


def _tcb_body(s_ref, stats_ref, dinv_ref, g_ref, be_ref, w_ref, out_ref):
    st = stats_ref[...]
    mu = st[0:1] / N
    var = st[1:2] / N - mu * mu
    h = jnp.maximum(
        (s_ref[...] - mu) * lax.rsqrt(var + EPSV) * g_ref[...] + be_ref[...], 0.0
    )
    out_ref[...] = dinv_ref[...] * jnp.dot(
        h, w_ref[...], preferred_element_type=jnp.float32
    )


def _tcb(s, stats, dinv, g, be, w):
    return pl.pallas_call(
        _tcb_body,
        grid=(NB,),
        in_specs=[
            pl.BlockSpec((BR, D), lambda i: (i, 0)),
            pl.BlockSpec((2, D), lambda i: (0, 0)),
            pl.BlockSpec((BR, D), lambda i: (i, 0)),
            pl.BlockSpec((1, D), lambda i: (0, 0)),
            pl.BlockSpec((1, D), lambda i: (0, 0)),
            pl.BlockSpec((D, D), lambda i: (0, 0)),
        ],
        out_specs=pl.BlockSpec((BR, D), lambda i: (i, 0)),
        out_shape=jax.ShapeDtypeStruct((N, D), jnp.float32),
    )(s, stats, dinv, g, be, w)


def _tcf_body(s_ref, stats_ref, g_ref, be_ref, bcol_ref, brow_ref, out_ref,
              sums_sc, cnts_sc, maxs_sc):
    i = pl.program_id(0)
    st = stats_ref[...]
    mu = st[0:1] / N
    var = st[1:2] / N - mu * mu
    h = jnp.maximum(
        (s_ref[...] - mu) * lax.rsqrt(var + EPSV) * g_ref[...] + be_ref[...], 0.0
    )

    @pl.when(i == 0)
    def _():
        sums_sc[...] = jnp.zeros_like(sums_sc)
        cnts_sc[...] = jnp.zeros_like(cnts_sc)
        maxs_sc[...] = jnp.full_like(maxs_sc, -jnp.inf)

    brow = brow_ref[0]
    gcol = lax.broadcasted_iota(jnp.int32, (G, 1), 0)
    oh = (brow == gcol).astype(jnp.float32)
    sums_sc[...] = sums_sc[...] + jnp.dot(oh, h, preferred_element_type=jnp.float32)
    cnts_sc[...] = cnts_sc[...] + jnp.sum(oh, axis=1, keepdims=True)

    bcol = bcol_ref[...]
    g_lo = jnp.min(bcol)
    g_hi = jnp.max(bcol)

    def gbody(g, carry):
        m = jnp.max(jnp.where(bcol == g, h, -jnp.inf), axis=0, keepdims=True)
        maxs_sc[pl.ds(g, 1), :] = jnp.maximum(maxs_sc[pl.ds(g, 1), :], m)
        return carry

    lax.fori_loop(g_lo, g_hi + 1, gbody, 0)

    @pl.when(i == NB - 1)
    def _():
        out_ref[...] = jnp.concatenate(
            [sums_sc[...] / jnp.maximum(cnts_sc[...], 1.0), maxs_sc[...]], axis=1
        )


def _tcf(s, stats, g, be, bcol, brow):
    return pl.pallas_call(
        _tcf_body,
        grid=(NB,),
        in_specs=[
            pl.BlockSpec((BR, D), lambda i: (i, 0)),
            pl.BlockSpec((2, D), lambda i: (0, 0)),
            pl.BlockSpec((1, D), lambda i: (0, 0)),
            pl.BlockSpec((1, D), lambda i: (0, 0)),
            pl.BlockSpec((BR, 1), lambda i: (i, 0)),
            pl.BlockSpec((1, 1, BR), lambda i: (i, 0, 0)),
        ],
        out_specs=pl.BlockSpec((G, 2 * D), lambda i: (0, 0)),
        out_shape=jax.ShapeDtypeStruct((G, 2 * D), jnp.float32),
        scratch_shapes=[
            pltpu.VMEM((G, D), jnp.float32),
            pltpu.VMEM((G, 1), jnp.float32),
            pltpu.VMEM((G, D), jnp.float32),
        ],
    )(s, stats, g, be, bcol, brow)


def kernel(x, edge_index, batch, W0, b0, gamma0, beta0, W1, b1, gamma1, beta1,
           W2, b2, gamma2, beta2):
    row = edge_index[0]
    col = edge_index[1]
    zeros_nd = jnp.zeros((N, D), jnp.float32)
    zeros_n16 = jnp.zeros((N, 16), jnp.float32)
    ones_ch16 = jnp.ones((CH, 16), jnp.float32)

    sc_degree, sc_agg = _sc_kernels()
    degp = sc_degree(col, ones_ch16, zeros_n16)
    y, dinvb = _tc0(degp, x, W0)

    Ws = [W1, W2]
    bs = [b0, b1, b2]
    gs = [gamma0, gamma1, gamma2]
    bes = [beta0, beta1, beta2]
    for i in range(3):
        acc = sc_agg(y, row, col, zeros_nd)
        s, stats = _tca(acc[0], acc[1], y, dinvb, bs[i].reshape(1, D))
        if i < 2:
            y = _tcb(s, stats, dinvb, gs[i].reshape(1, D), bes[i].reshape(1, D), Ws[i])
        else:
            out = _tcf(s, stats, gs[i].reshape(1, D), bes[i].reshape(1, D),
                       batch.reshape(N, 1), batch.reshape(NB, 1, BR))
    return out


# speedup vs baseline: 21.4951x; 1.9917x over previous
import functools

import jax
import jax.numpy as jnp
from jax import lax
from jax.experimental import pallas as pl
from jax.experimental.pallas import tpu as pltpu
from jax.experimental.pallas import tpu_sc as plsc

N = 10000
E = 320000
D = 128
G = 64
EPSV = 1e-5

NC = 2
NS = 16
NW = NC * NS
E_W = E // NW
CH = 80
NCH = E_W // CH
S0 = 624
S_LAST = N - S0 * (NS - 1)

NB = 5
BR = N // NB

def _sc_degree_body(col_hbm, ones_hbm, zeros_hbm, out_hbm, col_v, ones_v, acc):
    c = lax.axis_index("c")
    s = lax.axis_index("s")
    @pl.when(s < NS - 1)
    def _():
        pltpu.sync_copy(zeros_hbm.at[pl.ds(s * S0, S0)], acc.at[pl.ds(s * S0, S0)])

    @pl.when(s == NS - 1)
    def _():
        pltpu.sync_copy(zeros_hbm.at[pl.ds((NS - 1) * S0, S_LAST)],
                        acc.at[pl.ds((NS - 1) * S0, S_LAST)])

    pltpu.sync_copy(ones_hbm, ones_v)
    plsc.subcore_barrier()
    wid = c * NS + s

    def body(i, carry):
        eb = wid * E_W + i * CH
        pltpu.sync_copy(col_hbm.at[pl.ds(eb, CH)], col_v)
        pltpu.sync_copy(ones_v, acc.at[col_v], add=True)
        return carry

    lax.fori_loop(0, NCH, body, 0)
    plsc.subcore_barrier()

    @pl.when(s < NS - 1)
    def _():
        pltpu.sync_copy(acc.at[pl.ds(s * S0, S0)], out_hbm.at[c, pl.ds(s * S0, S0)])

    @pl.when(s == NS - 1)
    def _():
        pltpu.sync_copy(acc.at[pl.ds((NS - 1) * S0, S_LAST)],
                        out_hbm.at[c, pl.ds((NS - 1) * S0, S_LAST)])


def _sc_agg_body(y_hbm, row3_hbm, col3_hbm, zeros_hbm, out_hbm,
                 row_all, col_all, rows0, rows1, acc, sem0, sem1):
    c = lax.axis_index("c")
    s = lax.axis_index("s")
    wid = c * NS + s
    pltpu.sync_copy(row3_hbm.at[wid], row_all)
    pltpu.sync_copy(col3_hbm.at[wid], col_all)

    @pl.when(s < NS - 1)
    def _():
        pltpu.sync_copy(zeros_hbm.at[pl.ds(s * S0, S0)], acc.at[pl.ds(s * S0, S0)])

    @pl.when(s == NS - 1)
    def _():
        pltpu.sync_copy(zeros_hbm.at[pl.ds((NS - 1) * S0, S_LAST)],
                        acc.at[pl.ds((NS - 1) * S0, S_LAST)])

    pltpu.async_copy(y_hbm.at[row_all.at[0]], rows0, sem0)
    pltpu.async_copy(y_hbm.at[row_all.at[1]], rows1, sem1)
    plsc.subcore_barrier()

    def body(j, carry):
        i0 = 2 * j
        pltpu.make_async_copy(y_hbm.at[row_all.at[i0]], rows0, sem0).wait()
        pltpu.sync_copy(rows0, acc.at[col_all.at[i0]], add=True)

        @pl.when(i0 + 2 < NCH)
        def _():
            pltpu.async_copy(y_hbm.at[row_all.at[i0 + 2]], rows0, sem0)

        pltpu.make_async_copy(y_hbm.at[row_all.at[i0 + 1]], rows1, sem1).wait()
        pltpu.sync_copy(rows1, acc.at[col_all.at[i0 + 1]], add=True)

        @pl.when(i0 + 3 < NCH)
        def _():
            pltpu.async_copy(y_hbm.at[row_all.at[i0 + 3]], rows1, sem1)

        return carry

    lax.fori_loop(0, NCH // 2, body, 0)
    pltpu.make_async_copy(y_hbm.at[row_all.at[NCH - 1]], rows0, sem0).wait()
    pltpu.sync_copy(rows0, acc.at[col_all.at[NCH - 1]], add=True)
    plsc.subcore_barrier()

    @pl.when(s < NS - 1)
    def _():
        pltpu.sync_copy(acc.at[pl.ds(s * S0, S0)], out_hbm.at[c, pl.ds(s * S0, S0)])

    @pl.when(s == NS - 1)
    def _():
        pltpu.sync_copy(acc.at[pl.ds((NS - 1) * S0, S_LAST)],
                        out_hbm.at[c, pl.ds((NS - 1) * S0, S_LAST)])


@functools.cache
def _sc_kernels():
    mesh = plsc.VectorSubcoreMesh(
        core_axis_name="c", subcore_axis_name="s", num_cores=NC, num_subcores=NS
    )
    sc_degree = pl.kernel(
        _sc_degree_body,
        out_type=jax.ShapeDtypeStruct((NC, N, 16), jnp.float32),
        mesh=mesh,
        compiler_params=pltpu.CompilerParams(use_tc_tiling_on_sc=False),
        scratch_types=[
            pltpu.VMEM((CH,), jnp.int32),
            pltpu.VMEM((CH, 16), jnp.float32),
            pltpu.VMEM_SHARED((N, 16), jnp.float32),
        ],
    )
    sc_agg = pl.kernel(
        _sc_agg_body,
        out_type=jax.ShapeDtypeStruct((NC, N, D), jnp.float32),
        mesh=mesh,
        compiler_params=pltpu.CompilerParams(use_tc_tiling_on_sc=False),
        scratch_types=[
            pltpu.VMEM((NCH, CH), jnp.int32),
            pltpu.VMEM((NCH, CH), jnp.int32),
            pltpu.VMEM((CH, D), jnp.float32),
            pltpu.VMEM((CH, D), jnp.float32),
            pltpu.VMEM_SHARED((N, D), jnp.float32),
            pltpu.SemaphoreType.DMA,
            pltpu.SemaphoreType.DMA,
        ],
    )
    return sc_degree, sc_agg


def _tc0_body(degp_ref, x_ref, w_ref, y_ref, dinv_ref):
    dp = degp_ref[...]
    deg = dp[0, :, 0:1] + dp[1, :, 0:1] + 1.0
    dinv = lax.rsqrt(deg)
    dvb = jnp.broadcast_to(dinv, (BR, D))
    dinv_ref[...] = dvb
    y_ref[...] = dvb * jnp.dot(x_ref[...], w_ref[...], preferred_element_type=jnp.float32)


def _tc0(degp, x, w0):
    return pl.pallas_call(
        _tc0_body,
        grid=(NB,),
        in_specs=[
            pl.BlockSpec((NC, BR, 16), lambda i: (0, i, 0)),
            pl.BlockSpec((BR, D), lambda i: (i, 0)),
            pl.BlockSpec((D, D), lambda i: (0, 0)),
        ],
        out_specs=[
            pl.BlockSpec((BR, D), lambda i: (i, 0)),
            pl.BlockSpec((BR, D), lambda i: (i, 0)),
        ],
        out_shape=[
            jax.ShapeDtypeStruct((N, D), jnp.float32),
            jax.ShapeDtypeStruct((N, D), jnp.float32),
        ],
    )(degp, x, w0)


def _tca_body(a0_ref, a1_ref, y_ref, dinv_ref, b_ref, s_ref, stats_ref):
    i = pl.program_id(0)
    s = dinv_ref[...] * (a0_ref[...] + a1_ref[...] + y_ref[...]) + b_ref[...]
    s_ref[...] = s
    part = jnp.concatenate(
        [jnp.sum(s, axis=0, keepdims=True), jnp.sum(s * s, axis=0, keepdims=True)],
        axis=0,
    )

    @pl.when(i == 0)
    def _():
        stats_ref[...] = part

    @pl.when(i > 0)
    def _():
        stats_ref[...] = stats_ref[...] + part


def _tca(a0, a1, y, dinv, b):
    return pl.pallas_call(
        _tca_body,
        grid=(NB,),
        in_specs=[
            pl.BlockSpec((BR, D), lambda i: (i, 0)),
            pl.BlockSpec((BR, D), lambda i: (i, 0)),
            pl.BlockSpec((BR, D), lambda i: (i, 0)),
            pl.BlockSpec((BR, D), lambda i: (i, 0)),
            pl.BlockSpec((1, D), lambda i: (0, 0)),
        ],
        out_specs=[
            pl.BlockSpec((BR, D), lambda i: (i, 0)),
            pl.BlockSpec((2, D), lambda i: (0, 0)),
        ],
        out_shape=[
            jax.ShapeDtypeStruct((N, D), jnp.float32),
            jax.ShapeDtypeStruct((2, D), jnp.float32),
        ],
    )(a0, a1, y, dinv, b)


def _tcb_body(s_ref, stats_ref, dinv_ref, g_ref, be_ref, w_ref, out_ref):
    st = stats_ref[...]
    mu = st[0:1] / N
    var = st[1:2] / N - mu * mu
    h = jnp.maximum(
        (s_ref[...] - mu) * lax.rsqrt(var + EPSV) * g_ref[...] + be_ref[...], 0.0
    )
    out_ref[...] = dinv_ref[...] * jnp.dot(
        h, w_ref[...], preferred_element_type=jnp.float32
    )


def _tcb(s, stats, dinv, g, be, w):
    return pl.pallas_call(
        _tcb_body,
        grid=(NB,),
        in_specs=[
            pl.BlockSpec((BR, D), lambda i: (i, 0)),
            pl.BlockSpec((2, D), lambda i: (0, 0)),
            pl.BlockSpec((BR, D), lambda i: (i, 0)),
            pl.BlockSpec((1, D), lambda i: (0, 0)),
            pl.BlockSpec((1, D), lambda i: (0, 0)),
            pl.BlockSpec((D, D), lambda i: (0, 0)),
        ],
        out_specs=pl.BlockSpec((BR, D), lambda i: (i, 0)),
        out_shape=jax.ShapeDtypeStruct((N, D), jnp.float32),
    )(s, stats, dinv, g, be, w)


def _tcf_body(s_ref, stats_ref, g_ref, be_ref, bcol_ref, brow_ref, out_ref,
              sums_sc, cnts_sc, maxs_sc):
    i = pl.program_id(0)
    st = stats_ref[...]
    mu = st[0:1] / N
    var = st[1:2] / N - mu * mu
    h = jnp.maximum(
        (s_ref[...] - mu) * lax.rsqrt(var + EPSV) * g_ref[...] + be_ref[...], 0.0
    )

    @pl.when(i == 0)
    def _():
        sums_sc[...] = jnp.zeros_like(sums_sc)
        cnts_sc[...] = jnp.zeros_like(cnts_sc)
        maxs_sc[...] = jnp.full_like(maxs_sc, -jnp.inf)

    brow = brow_ref[0]
    gcol = lax.broadcasted_iota(jnp.int32, (G, 1), 0)
    oh = (brow == gcol).astype(jnp.float32)
    sums_sc[...] = sums_sc[...] + jnp.dot(oh, h, preferred_element_type=jnp.float32)
    cnts_sc[...] = cnts_sc[...] + jnp.sum(oh, axis=1, keepdims=True)

    bcol = bcol_ref[...]
    g_lo = jnp.min(bcol)
    g_hi = jnp.max(bcol)

    def gbody(g, carry):
        m = jnp.max(jnp.where(bcol == g, h, -jnp.inf), axis=0, keepdims=True)
        maxs_sc[pl.ds(g, 1), :] = jnp.maximum(maxs_sc[pl.ds(g, 1), :], m)
        return carry

    lax.fori_loop(g_lo, g_hi + 1, gbody, 0)

    @pl.when(i == NB - 1)
    def _():
        out_ref[...] = jnp.concatenate(
            [sums_sc[...] / jnp.maximum(cnts_sc[...], 1.0), maxs_sc[...]], axis=1
        )


def _tcf(s, stats, g, be, bcol, brow):
    return pl.pallas_call(
        _tcf_body,
        grid=(NB,),
        in_specs=[
            pl.BlockSpec((BR, D), lambda i: (i, 0)),
            pl.BlockSpec((2, D), lambda i: (0, 0)),
            pl.BlockSpec((1, D), lambda i: (0, 0)),
            pl.BlockSpec((1, D), lambda i: (0, 0)),
            pl.BlockSpec((BR, 1), lambda i: (i, 0)),
            pl.BlockSpec((1, 1, BR), lambda i: (i, 0, 0)),
        ],
        out_specs=pl.BlockSpec((G, 2 * D), lambda i: (0, 0)),
        out_shape=jax.ShapeDtypeStruct((G, 2 * D), jnp.float32),
        scratch_shapes=[
            pltpu.VMEM((G, D), jnp.float32),
            pltpu.VMEM((G, 1), jnp.float32),
            pltpu.VMEM((G, D), jnp.float32),
        ],
    )(s, stats, g, be, bcol, brow)


def kernel(x, edge_index, batch, W0, b0, gamma0, beta0, W1, b1, gamma1, beta1,
           W2, b2, gamma2, beta2):
    row = edge_index[0]
    col = edge_index[1]
    row3 = row.reshape(NW, NCH, CH)
    col3 = col.reshape(NW, NCH, CH)
    zeros_nd = jnp.zeros((N, D), jnp.float32)
    zeros_n16 = jnp.zeros((N, 16), jnp.float32)
    ones_ch16 = jnp.ones((CH, 16), jnp.float32)

    sc_degree, sc_agg = _sc_kernels()
    degp = sc_degree(col, ones_ch16, zeros_n16)
    y, dinvb = _tc0(degp, x, W0)

    Ws = [W1, W2]
    bs = [b0, b1, b2]
    gs = [gamma0, gamma1, gamma2]
    bes = [beta0, beta1, beta2]
    for i in range(3):
        acc = sc_agg(y, row3, col3, zeros_nd)
        s, stats = _tca(acc[0], acc[1], y, dinvb, bs[i].reshape(1, D))
        if i < 2:
            y = _tcb(s, stats, dinvb, gs[i].reshape(1, D), bes[i].reshape(1, D), Ws[i])
        else:
            out = _tcf(s, stats, gs[i].reshape(1, D), bes[i].reshape(1, D),
                       batch.reshape(N, 1), batch.reshape(NB, 1, BR))
    return out


# speedup vs baseline: 27.0525x; 1.2585x over previous
import functools

import jax
import jax.numpy as jnp
from jax import lax
from jax.experimental import pallas as pl
from jax.experimental.pallas import tpu as pltpu
from jax.experimental.pallas import tpu_sc as plsc

N = 10000
E = 320000
D = 128
G = 64
EPSV = 1e-5

NC = 2
NS = 16
NW = NC * NS
E_W = E // NW
CH = 80
NCH = E_W // CH
S0 = 624
S_LAST = N - S0 * (NS - 1)
NBUF = 3

NB = 5
BR = N // NB

def _sc_degree_body(col3_hbm, ones_hbm, zeros_hbm, out_hbm, col_all, ones_v, acc):
    c = lax.axis_index("c")
    s = lax.axis_index("s")
    wid = c * NS + s
    pltpu.sync_copy(col3_hbm.at[wid], col_all)

    @pl.when(s < NS - 1)
    def _():
        pltpu.sync_copy(zeros_hbm.at[pl.ds(s * S0, S0)], acc.at[pl.ds(s * S0, S0)])

    @pl.when(s == NS - 1)
    def _():
        pltpu.sync_copy(zeros_hbm.at[pl.ds((NS - 1) * S0, S_LAST)],
                        acc.at[pl.ds((NS - 1) * S0, S_LAST)])

    pltpu.sync_copy(ones_hbm, ones_v)
    plsc.subcore_barrier()

    def body(i, carry):
        pltpu.sync_copy(ones_v, acc.at[col_all.at[i]], add=True)
        return carry

    lax.fori_loop(0, NCH, body, 0)
    plsc.subcore_barrier()

    @pl.when(s < NS - 1)
    def _():
        pltpu.sync_copy(acc.at[pl.ds(s * S0, S0)], out_hbm.at[c, pl.ds(s * S0, S0)])

    @pl.when(s == NS - 1)
    def _():
        pltpu.sync_copy(acc.at[pl.ds((NS - 1) * S0, S_LAST)],
                        out_hbm.at[c, pl.ds((NS - 1) * S0, S_LAST)])


def _sc_agg_body(y_hbm, row3_hbm, col3_hbm, zeros_hbm, out_hbm,
                 row_all, col_all, rows0, rows1, rows2, acc, sem0, sem1, sem2):
    c = lax.axis_index("c")
    s = lax.axis_index("s")
    wid = c * NS + s
    pltpu.sync_copy(row3_hbm.at[wid], row_all)
    pltpu.sync_copy(col3_hbm.at[wid], col_all)

    @pl.when(s < NS - 1)
    def _():
        pltpu.sync_copy(zeros_hbm.at[pl.ds(s * S0, S0)], acc.at[pl.ds(s * S0, S0)])

    @pl.when(s == NS - 1)
    def _():
        pltpu.sync_copy(zeros_hbm.at[pl.ds((NS - 1) * S0, S_LAST)],
                        acc.at[pl.ds((NS - 1) * S0, S_LAST)])

    bufs = (rows0, rows1, rows2)
    sems = (sem0, sem1, sem2)
    for b in range(NBUF):
        pltpu.async_copy(y_hbm.at[row_all.at[b]], bufs[b], sems[b])
    plsc.subcore_barrier()

    def body(j, carry):
        for b in range(NBUF):
            i = j * NBUF + b

            @pl.when(i < NCH)
            def _(i=i, b=b):
                pltpu.make_async_copy(y_hbm.at[row_all.at[i]], bufs[b], sems[b]).wait()
                pltpu.sync_copy(bufs[b], acc.at[col_all.at[i]], add=True)

                @pl.when(i + NBUF < NCH)
                def _():
                    pltpu.async_copy(y_hbm.at[row_all.at[i + NBUF]], bufs[b], sems[b])

        return carry

    lax.fori_loop(0, (NCH + NBUF - 1) // NBUF, body, 0)
    plsc.subcore_barrier()

    @pl.when(s < NS - 1)
    def _():
        pltpu.sync_copy(acc.at[pl.ds(s * S0, S0)], out_hbm.at[c, pl.ds(s * S0, S0)])

    @pl.when(s == NS - 1)
    def _():
        pltpu.sync_copy(acc.at[pl.ds((NS - 1) * S0, S_LAST)],
                        out_hbm.at[c, pl.ds((NS - 1) * S0, S_LAST)])


@functools.cache
def _sc_kernels():
    mesh = plsc.VectorSubcoreMesh(
        core_axis_name="c", subcore_axis_name="s", num_cores=NC, num_subcores=NS
    )
    sc_degree = pl.kernel(
        _sc_degree_body,
        out_type=jax.ShapeDtypeStruct((NC, N, 16), jnp.float32),
        mesh=mesh,
        compiler_params=pltpu.CompilerParams(use_tc_tiling_on_sc=False),
        scratch_types=[
            pltpu.VMEM((NCH, CH), jnp.int32),
            pltpu.VMEM((CH, 16), jnp.float32),
            pltpu.VMEM_SHARED((N, 16), jnp.float32),
        ],
    )
    sc_agg = pl.kernel(
        _sc_agg_body,
        out_type=jax.ShapeDtypeStruct((NC, N, D), jnp.float32),
        mesh=mesh,
        compiler_params=pltpu.CompilerParams(use_tc_tiling_on_sc=False),
        scratch_types=[
            pltpu.VMEM((NCH, CH), jnp.int32),
            pltpu.VMEM((NCH, CH), jnp.int32),
            pltpu.VMEM((CH, D), jnp.float32),
            pltpu.VMEM((CH, D), jnp.float32),
            pltpu.VMEM((CH, D), jnp.float32),
            pltpu.VMEM_SHARED((N, D), jnp.float32),
            pltpu.SemaphoreType.DMA,
            pltpu.SemaphoreType.DMA,
            pltpu.SemaphoreType.DMA,
        ],
    )
    return sc_degree, sc_agg


def _tc0_body(degp_ref, x_ref, w_ref, y_ref, dinv_ref):
    dp = degp_ref[...]
    deg = dp[0, :, 0:1] + dp[1, :, 0:1] + 1.0
    dinv = lax.rsqrt(deg)
    dvb = jnp.broadcast_to(dinv, (BR, D))
    dinv_ref[...] = dvb
    y_ref[...] = dvb * jnp.dot(x_ref[...], w_ref[...], preferred_element_type=jnp.float32)


def _tc0(degp, x, w0):
    return pl.pallas_call(
        _tc0_body,
        grid=(NB,),
        in_specs=[
            pl.BlockSpec((NC, BR, 16), lambda i: (0, i, 0)),
            pl.BlockSpec((BR, D), lambda i: (i, 0)),
            pl.BlockSpec((D, D), lambda i: (0, 0)),
        ],
        out_specs=[
            pl.BlockSpec((BR, D), lambda i: (i, 0)),
            pl.BlockSpec((BR, D), lambda i: (i, 0)),
        ],
        out_shape=[
            jax.ShapeDtypeStruct((N, D), jnp.float32),
            jax.ShapeDtypeStruct((N, D), jnp.float32),
        ],
    )(degp, x, w0)


def _tca_body(a0_ref, a1_ref, y_ref, dinv_ref, b_ref, s_ref, stats_ref):
    i = pl.program_id(0)
    s = dinv_ref[...] * (a0_ref[...] + a1_ref[...] + y_ref[...]) + b_ref[...]
    s_ref[...] = s
    part = jnp.concatenate(
        [jnp.sum(s, axis=0, keepdims=True), jnp.sum(s * s, axis=0, keepdims=True)],
        axis=0,
    )

    @pl.when(i == 0)
    def _():
        stats_ref[...] = part

    @pl.when(i > 0)
    def _():
        stats_ref[...] = stats_ref[...] + part


def _tca(a0, a1, y, dinv, b):
    return pl.pallas_call(
        _tca_body,
        grid=(NB,),
        in_specs=[
            pl.BlockSpec((BR, D), lambda i: (i, 0)),
            pl.BlockSpec((BR, D), lambda i: (i, 0)),
            pl.BlockSpec((BR, D), lambda i: (i, 0)),
            pl.BlockSpec((BR, D), lambda i: (i, 0)),
            pl.BlockSpec((1, D), lambda i: (0, 0)),
        ],
        out_specs=[
            pl.BlockSpec((BR, D), lambda i: (i, 0)),
            pl.BlockSpec((2, D), lambda i: (0, 0)),
        ],
        out_shape=[
            jax.ShapeDtypeStruct((N, D), jnp.float32),
            jax.ShapeDtypeStruct((2, D), jnp.float32),
        ],
    )(a0, a1, y, dinv, b)


def _tcb_body(s_ref, stats_ref, dinv_ref, g_ref, be_ref, w_ref, out_ref):
    st = stats_ref[...]
    mu = st[0:1] / N
    var = st[1:2] / N - mu * mu
    h = jnp.maximum(
        (s_ref[...] - mu) * lax.rsqrt(var + EPSV) * g_ref[...] + be_ref[...], 0.0
    )
    out_ref[...] = dinv_ref[...] * jnp.dot(
        h, w_ref[...], preferred_element_type=jnp.float32
    )


def _tcb(s, stats, dinv, g, be, w):
    return pl.pallas_call(
        _tcb_body,
        grid=(NB,),
        in_specs=[
            pl.BlockSpec((BR, D), lambda i: (i, 0)),
            pl.BlockSpec((2, D), lambda i: (0, 0)),
            pl.BlockSpec((BR, D), lambda i: (i, 0)),
            pl.BlockSpec((1, D), lambda i: (0, 0)),
            pl.BlockSpec((1, D), lambda i: (0, 0)),
            pl.BlockSpec((D, D), lambda i: (0, 0)),
        ],
        out_specs=pl.BlockSpec((BR, D), lambda i: (i, 0)),
        out_shape=jax.ShapeDtypeStruct((N, D), jnp.float32),
    )(s, stats, dinv, g, be, w)


def _tcf_body(s_ref, stats_ref, g_ref, be_ref, bcol_ref, brow_ref, out_ref,
              sums_sc, cnts_sc, maxs_sc):
    i = pl.program_id(0)
    st = stats_ref[...]
    mu = st[0:1] / N
    var = st[1:2] / N - mu * mu
    h = jnp.maximum(
        (s_ref[...] - mu) * lax.rsqrt(var + EPSV) * g_ref[...] + be_ref[...], 0.0
    )

    @pl.when(i == 0)
    def _():
        sums_sc[...] = jnp.zeros_like(sums_sc)
        cnts_sc[...] = jnp.zeros_like(cnts_sc)
        maxs_sc[...] = jnp.full_like(maxs_sc, -jnp.inf)

    brow = brow_ref[0]
    gcol = lax.broadcasted_iota(jnp.int32, (G, 1), 0)
    oh = (brow == gcol).astype(jnp.float32)
    sums_sc[...] = sums_sc[...] + jnp.dot(oh, h, preferred_element_type=jnp.float32)
    cnts_sc[...] = cnts_sc[...] + jnp.sum(oh, axis=1, keepdims=True)

    bcol = bcol_ref[...]
    g_lo = jnp.min(bcol)
    g_hi = jnp.max(bcol)

    def gbody(g, carry):
        m = jnp.max(jnp.where(bcol == g, h, -jnp.inf), axis=0, keepdims=True)
        maxs_sc[pl.ds(g, 1), :] = jnp.maximum(maxs_sc[pl.ds(g, 1), :], m)
        return carry

    lax.fori_loop(g_lo, g_hi + 1, gbody, 0)

    @pl.when(i == NB - 1)
    def _():
        out_ref[...] = jnp.concatenate(
            [sums_sc[...] / jnp.maximum(cnts_sc[...], 1.0), maxs_sc[...]], axis=1
        )


def _tcf(s, stats, g, be, bcol, brow):
    return pl.pallas_call(
        _tcf_body,
        grid=(NB,),
        in_specs=[
            pl.BlockSpec((BR, D), lambda i: (i, 0)),
            pl.BlockSpec((2, D), lambda i: (0, 0)),
            pl.BlockSpec((1, D), lambda i: (0, 0)),
            pl.BlockSpec((1, D), lambda i: (0, 0)),
            pl.BlockSpec((BR, 1), lambda i: (i, 0)),
            pl.BlockSpec((1, 1, BR), lambda i: (i, 0, 0)),
        ],
        out_specs=pl.BlockSpec((G, 2 * D), lambda i: (0, 0)),
        out_shape=jax.ShapeDtypeStruct((G, 2 * D), jnp.float32),
        scratch_shapes=[
            pltpu.VMEM((G, D), jnp.float32),
            pltpu.VMEM((G, 1), jnp.float32),
            pltpu.VMEM((G, D), jnp.float32),
        ],
    )(s, stats, g, be, bcol, brow)


def kernel(x, edge_index, batch, W0, b0, gamma0, beta0, W1, b1, gamma1, beta1,
           W2, b2, gamma2, beta2):
    row = edge_index[0]
    col = edge_index[1]
    row3 = row.reshape(NW, NCH, CH)
    col3 = col.reshape(NW, NCH, CH)
    zeros_nd = jnp.zeros((N, D), jnp.float32)
    zeros_n16 = jnp.zeros((N, 16), jnp.float32)
    ones_ch16 = jnp.ones((CH, 16), jnp.float32)

    sc_degree, sc_agg = _sc_kernels()
    degp = sc_degree(col3, ones_ch16, zeros_n16)
    y, dinvb = _tc0(degp, x, W0)

    Ws = [W1, W2]
    bs = [b0, b1, b2]
    gs = [gamma0, gamma1, gamma2]
    bes = [beta0, beta1, beta2]
    for i in range(3):
        acc = sc_agg(y, row3, col3, zeros_nd)
        s, stats = _tca(acc[0], acc[1], y, dinvb, bs[i].reshape(1, D))
        if i < 2:
            y = _tcb(s, stats, dinvb, gs[i].reshape(1, D), bes[i].reshape(1, D), Ws[i])
        else:
            out = _tcf(s, stats, gs[i].reshape(1, D), bes[i].reshape(1, D),
                       batch.reshape(N, 1), batch.reshape(NB, 1, BR))
    return out


# speedup vs baseline: 27.7715x; 1.0266x over previous
import functools

import jax
import jax.numpy as jnp
from jax import lax
from jax.experimental import pallas as pl
from jax.experimental.pallas import tpu as pltpu
from jax.experimental.pallas import tpu_sc as plsc

N = 10000
E = 320000
D = 128
G = 64
EPSV = 1e-5

NC = 2
NS = 16
NW = NC * NS
E_W = E // NW
CH = 80
NCH = E_W // CH
S0 = 624
S_LAST = N - S0 * (NS - 1)
NBUF = 3

NB = 5
BR = N // NB

def _sc_degree_body(col3_hbm, ones_hbm, zeros_hbm, out_hbm, col_all, ones_v, acc):
    c = lax.axis_index("c")
    s = lax.axis_index("s")
    wid = c * NS + s
    pltpu.sync_copy(col3_hbm.at[wid], col_all)

    @pl.when(s < NS - 1)
    def _():
        pltpu.sync_copy(zeros_hbm.at[pl.ds(s * S0, S0)], acc.at[pl.ds(s * S0, S0)])

    @pl.when(s == NS - 1)
    def _():
        pltpu.sync_copy(zeros_hbm.at[pl.ds((NS - 1) * S0, S_LAST)],
                        acc.at[pl.ds((NS - 1) * S0, S_LAST)])

    pltpu.sync_copy(ones_hbm, ones_v)
    plsc.subcore_barrier()

    def body(i, carry):
        pltpu.sync_copy(ones_v, acc.at[col_all.at[i]], add=True)
        return carry

    lax.fori_loop(0, NCH, body, 0)
    plsc.subcore_barrier()

    @pl.when(s < NS - 1)
    def _():
        pltpu.sync_copy(acc.at[pl.ds(s * S0, S0)], out_hbm.at[c, pl.ds(s * S0, S0)])

    @pl.when(s == NS - 1)
    def _():
        pltpu.sync_copy(acc.at[pl.ds((NS - 1) * S0, S_LAST)],
                        out_hbm.at[c, pl.ds((NS - 1) * S0, S_LAST)])


def _sc_agg_body(y_hbm, row3_hbm, col3_hbm, zeros_hbm, out_hbm,
                 row_all, col_all, rows0, rows1, rows2, acc, sem0, sem1, sem2):
    c = lax.axis_index("c")
    s = lax.axis_index("s")
    wid = c * NS + s
    pltpu.sync_copy(row3_hbm.at[wid], row_all)
    pltpu.sync_copy(col3_hbm.at[wid], col_all)

    @pl.when(s < NS - 1)
    def _():
        pltpu.sync_copy(zeros_hbm.at[pl.ds(s * S0, S0)], acc.at[pl.ds(s * S0, S0)])

    @pl.when(s == NS - 1)
    def _():
        pltpu.sync_copy(zeros_hbm.at[pl.ds((NS - 1) * S0, S_LAST)],
                        acc.at[pl.ds((NS - 1) * S0, S_LAST)])

    bufs = (rows0, rows1, rows2)
    sems = (sem0, sem1, sem2)
    for b in range(NBUF):
        pltpu.async_copy(y_hbm.at[row_all.at[b]], bufs[b], sems[b])
    plsc.subcore_barrier()

    def body(j, carry):
        for b in range(NBUF):
            i = j * NBUF + b

            @pl.when(i < NCH)
            def _(i=i, b=b):
                pltpu.make_async_copy(y_hbm.at[row_all.at[i]], bufs[b], sems[b]).wait()
                pltpu.sync_copy(bufs[b], acc.at[col_all.at[i]], add=True)

                @pl.when(i + NBUF < NCH)
                def _():
                    pltpu.async_copy(y_hbm.at[row_all.at[i + NBUF]], bufs[b], sems[b])

        return carry

    lax.fori_loop(0, (NCH + NBUF - 1) // NBUF, body, 0)
    plsc.subcore_barrier()

    @pl.when(s < NS - 1)
    def _():
        pltpu.sync_copy(acc.at[pl.ds(s * S0, S0)], out_hbm.at[c, pl.ds(s * S0, S0)])

    @pl.when(s == NS - 1)
    def _():
        pltpu.sync_copy(acc.at[pl.ds((NS - 1) * S0, S_LAST)],
                        out_hbm.at[c, pl.ds((NS - 1) * S0, S_LAST)])


@functools.cache
def _sc_kernels():
    mesh = plsc.VectorSubcoreMesh(
        core_axis_name="c", subcore_axis_name="s", num_cores=NC, num_subcores=NS
    )
    sc_degree = pl.kernel(
        _sc_degree_body,
        out_type=jax.ShapeDtypeStruct((NC, N, 16), jnp.float32),
        mesh=mesh,
        compiler_params=pltpu.CompilerParams(use_tc_tiling_on_sc=False),
        scratch_types=[
            pltpu.VMEM((NCH, CH), jnp.int32),
            pltpu.VMEM((CH, 16), jnp.float32),
            pltpu.VMEM_SHARED((N, 16), jnp.float32),
        ],
    )
    sc_agg = pl.kernel(
        _sc_agg_body,
        out_type=jax.ShapeDtypeStruct((NC, N, D), jnp.float32),
        mesh=mesh,
        compiler_params=pltpu.CompilerParams(use_tc_tiling_on_sc=False),
        scratch_types=[
            pltpu.VMEM((NCH, CH), jnp.int32),
            pltpu.VMEM((NCH, CH), jnp.int32),
            pltpu.VMEM((CH, D), jnp.float32),
            pltpu.VMEM((CH, D), jnp.float32),
            pltpu.VMEM((CH, D), jnp.float32),
            pltpu.VMEM_SHARED((N, D), jnp.float32),
            pltpu.SemaphoreType.DMA,
            pltpu.SemaphoreType.DMA,
            pltpu.SemaphoreType.DMA,
        ],
    )
    return sc_degree, sc_agg


def _tc0_body(degp_ref, x_ref, w_ref, y_ref, dinv_ref):
    dp = degp_ref[...]
    deg = dp[0, :, 0:1] + dp[1, :, 0:1] + 1.0
    dinv = lax.rsqrt(deg)
    dvb = jnp.broadcast_to(dinv, (BR, D))
    dinv_ref[...] = dvb
    y_ref[...] = dvb * jnp.dot(x_ref[...], w_ref[...], preferred_element_type=jnp.float32)


def _tc0(degp, x, w0):
    return pl.pallas_call(
        _tc0_body,
        grid=(NB,),
        in_specs=[
            pl.BlockSpec((NC, BR, 16), lambda i: (0, i, 0)),
            pl.BlockSpec((BR, D), lambda i: (i, 0)),
            pl.BlockSpec((D, D), lambda i: (0, 0)),
        ],
        out_specs=[
            pl.BlockSpec((BR, D), lambda i: (i, 0)),
            pl.BlockSpec((BR, D), lambda i: (i, 0)),
        ],
        out_shape=[
            jax.ShapeDtypeStruct((N, D), jnp.float32),
            jax.ShapeDtypeStruct((N, D), jnp.float32),
        ],
    )(degp, x, w0)


def _tca_body(a0_ref, a1_ref, y_ref, dinv_ref, b_ref, s_ref, stats_ref):
    i = pl.program_id(0)
    s = dinv_ref[...] * (a0_ref[...] + a1_ref[...] + y_ref[...]) + b_ref[...]
    s_ref[...] = s
    part = jnp.concatenate(
        [jnp.sum(s, axis=0, keepdims=True), jnp.sum(s * s, axis=0, keepdims=True)],
        axis=0,
    )

    @pl.when(i == 0)
    def _():
        stats_ref[...] = part

    @pl.when(i > 0)
    def _():
        stats_ref[...] = stats_ref[...] + part


def _tca(a0, a1, y, dinv, b):
    return pl.pallas_call(
        _tca_body,
        grid=(NB,),
        in_specs=[
            pl.BlockSpec((BR, D), lambda i: (i, 0)),
            pl.BlockSpec((BR, D), lambda i: (i, 0)),
            pl.BlockSpec((BR, D), lambda i: (i, 0)),
            pl.BlockSpec((BR, D), lambda i: (i, 0)),
            pl.BlockSpec((1, D), lambda i: (0, 0)),
        ],
        out_specs=[
            pl.BlockSpec((BR, D), lambda i: (i, 0)),
            pl.BlockSpec((2, D), lambda i: (0, 0)),
        ],
        out_shape=[
            jax.ShapeDtypeStruct((N, D), jnp.float32),
            jax.ShapeDtypeStruct((2, D), jnp.float32),
        ],
    )(a0, a1, y, dinv, b)


def _tcb_body(s_ref, stats_ref, dinv_ref, g_ref, be_ref, w_ref, out_ref):
    st = stats_ref[...]
    mu = st[0:1] / N
    var = st[1:2] / N - mu * mu
    h = jnp.maximum(
        (s_ref[...] - mu) * lax.rsqrt(var + EPSV) * g_ref[...] + be_ref[...], 0.0
    )
    out_ref[...] = dinv_ref[...] * jnp.dot(
        h, w_ref[...], preferred_element_type=jnp.float32
    )


def _tcb(s, stats, dinv, g, be, w):
    return pl.pallas_call(
        _tcb_body,
        grid=(NB,),
        in_specs=[
            pl.BlockSpec((BR, D), lambda i: (i, 0)),
            pl.BlockSpec((2, D), lambda i: (0, 0)),
            pl.BlockSpec((BR, D), lambda i: (i, 0)),
            pl.BlockSpec((1, D), lambda i: (0, 0)),
            pl.BlockSpec((1, D), lambda i: (0, 0)),
            pl.BlockSpec((D, D), lambda i: (0, 0)),
        ],
        out_specs=pl.BlockSpec((BR, D), lambda i: (i, 0)),
        out_shape=jax.ShapeDtypeStruct((N, D), jnp.float32),
    )(s, stats, dinv, g, be, w)


def _tcab_body(a0_ref, a1_ref, y_ref, dinv_ref, b_ref, g_ref, be_ref, w_ref,
               out_ref, s_buf, stats_sc):
    i = pl.program_id(0)

    @pl.when(i < NB)
    def _():
        s = dinv_ref[...] * (a0_ref[...] + a1_ref[...] + y_ref[...]) + b_ref[...]
        s_buf[pl.ds(i * BR, BR), :] = s
        part = jnp.concatenate(
            [jnp.sum(s, axis=0, keepdims=True), jnp.sum(s * s, axis=0, keepdims=True)],
            axis=0,
        )

        @pl.when(i == 0)
        def _():
            stats_sc[...] = part

        @pl.when(i > 0)
        def _():
            stats_sc[...] = stats_sc[...] + part

    @pl.when(i >= NB)
    def _():
        st = stats_sc[...]
        mu = st[0:1] / N
        var = st[1:2] / N - mu * mu
        sblk = s_buf[pl.ds((i - NB) * BR, BR), :]
        h = jnp.maximum(
            (sblk - mu) * lax.rsqrt(var + EPSV) * g_ref[...] + be_ref[...], 0.0
        )
        out_ref[...] = dinv_ref[...] * jnp.dot(
            h, w_ref[...], preferred_element_type=jnp.float32
        )


def _tcab(a0, a1, y, dinv, b, g, be, w):
    ph0 = lambda i: (jnp.where(i < NB, i, NB - 1), 0)
    both = lambda i: (jnp.where(i < NB, i, i - NB), 0)
    return pl.pallas_call(
        _tcab_body,
        grid=(2 * NB,),
        in_specs=[
            pl.BlockSpec((BR, D), ph0),
            pl.BlockSpec((BR, D), ph0),
            pl.BlockSpec((BR, D), ph0),
            pl.BlockSpec((BR, D), both),
            pl.BlockSpec((1, D), lambda i: (0, 0)),
            pl.BlockSpec((1, D), lambda i: (0, 0)),
            pl.BlockSpec((1, D), lambda i: (0, 0)),
            pl.BlockSpec((D, D), lambda i: (0, 0)),
        ],
        out_specs=pl.BlockSpec((BR, D), lambda i: (jnp.where(i < NB, 0, i - NB), 0)),
        out_shape=jax.ShapeDtypeStruct((N, D), jnp.float32),
        scratch_shapes=[
            pltpu.VMEM((N, D), jnp.float32),
            pltpu.VMEM((2, D), jnp.float32),
        ],
    )(a0, a1, y, dinv, b, g, be, w)


def _tcaf_body(a0_ref, a1_ref, y_ref, dinv_ref, b_ref, g_ref, be_ref,
               bcol_ref, brow_ref, out_ref, s_buf, stats_sc, sums_sc, cnts_sc, maxs_sc):
    i = pl.program_id(0)

    @pl.when(i < NB)
    def _():
        s = dinv_ref[...] * (a0_ref[...] + a1_ref[...] + y_ref[...]) + b_ref[...]
        s_buf[pl.ds(i * BR, BR), :] = s
        part = jnp.concatenate(
            [jnp.sum(s, axis=0, keepdims=True), jnp.sum(s * s, axis=0, keepdims=True)],
            axis=0,
        )

        @pl.when(i == 0)
        def _():
            stats_sc[...] = part

        @pl.when(i > 0)
        def _():
            stats_sc[...] = stats_sc[...] + part

    @pl.when(i >= NB)
    def _():
        st = stats_sc[...]
        mu = st[0:1] / N
        var = st[1:2] / N - mu * mu
        sblk = s_buf[pl.ds((i - NB) * BR, BR), :]
        h = jnp.maximum(
            (sblk - mu) * lax.rsqrt(var + EPSV) * g_ref[...] + be_ref[...], 0.0
        )

        @pl.when(i == NB)
        def _():
            sums_sc[...] = jnp.zeros_like(sums_sc)
            cnts_sc[...] = jnp.zeros_like(cnts_sc)
            maxs_sc[...] = jnp.full_like(maxs_sc, -jnp.inf)

        brow = brow_ref[0]
        gcol = lax.broadcasted_iota(jnp.int32, (G, 1), 0)
        oh = (brow == gcol).astype(jnp.float32)
        sums_sc[...] = sums_sc[...] + jnp.dot(oh, h, preferred_element_type=jnp.float32)
        cnts_sc[...] = cnts_sc[...] + jnp.sum(oh, axis=1, keepdims=True)

        bcol = bcol_ref[...]
        g_lo = jnp.min(bcol)
        g_hi = jnp.max(bcol)

        def gbody(gg, carry):
            m = jnp.max(jnp.where(bcol == gg, h, -jnp.inf), axis=0, keepdims=True)
            maxs_sc[pl.ds(gg, 1), :] = jnp.maximum(maxs_sc[pl.ds(gg, 1), :], m)
            return carry

        lax.fori_loop(g_lo, g_hi + 1, gbody, 0)

        @pl.when(i == 2 * NB - 1)
        def _():
            out_ref[...] = jnp.concatenate(
                [sums_sc[...] / jnp.maximum(cnts_sc[...], 1.0), maxs_sc[...]], axis=1
            )


def _tcaf(a0, a1, y, dinv, b, g, be, bcol, brow):
    ph0 = lambda i: (jnp.where(i < NB, i, NB - 1), 0)
    ph1 = lambda i: (jnp.where(i < NB, 0, i - NB), 0)
    return pl.pallas_call(
        _tcaf_body,
        grid=(2 * NB,),
        in_specs=[
            pl.BlockSpec((BR, D), ph0),
            pl.BlockSpec((BR, D), ph0),
            pl.BlockSpec((BR, D), ph0),
            pl.BlockSpec((BR, D), ph0),
            pl.BlockSpec((1, D), lambda i: (0, 0)),
            pl.BlockSpec((1, D), lambda i: (0, 0)),
            pl.BlockSpec((1, D), lambda i: (0, 0)),
            pl.BlockSpec((BR, 1), ph1),
            pl.BlockSpec((1, 1, BR), lambda i: (jnp.where(i < NB, 0, i - NB), 0, 0)),
        ],
        out_specs=pl.BlockSpec((G, 2 * D), lambda i: (0, 0)),
        out_shape=jax.ShapeDtypeStruct((G, 2 * D), jnp.float32),
        scratch_shapes=[
            pltpu.VMEM((N, D), jnp.float32),
            pltpu.VMEM((2, D), jnp.float32),
            pltpu.VMEM((G, D), jnp.float32),
            pltpu.VMEM((G, 1), jnp.float32),
            pltpu.VMEM((G, D), jnp.float32),
        ],
    )(a0, a1, y, dinv, b, g, be, bcol, brow)


def _tcf_body(s_ref, stats_ref, g_ref, be_ref, bcol_ref, brow_ref, out_ref,
              sums_sc, cnts_sc, maxs_sc):
    i = pl.program_id(0)
    st = stats_ref[...]
    mu = st[0:1] / N
    var = st[1:2] / N - mu * mu
    h = jnp.maximum(
        (s_ref[...] - mu) * lax.rsqrt(var + EPSV) * g_ref[...] + be_ref[...], 0.0
    )

    @pl.when(i == 0)
    def _():
        sums_sc[...] = jnp.zeros_like(sums_sc)
        cnts_sc[...] = jnp.zeros_like(cnts_sc)
        maxs_sc[...] = jnp.full_like(maxs_sc, -jnp.inf)

    brow = brow_ref[0]
    gcol = lax.broadcasted_iota(jnp.int32, (G, 1), 0)
    oh = (brow == gcol).astype(jnp.float32)
    sums_sc[...] = sums_sc[...] + jnp.dot(oh, h, preferred_element_type=jnp.float32)
    cnts_sc[...] = cnts_sc[...] + jnp.sum(oh, axis=1, keepdims=True)

    bcol = bcol_ref[...]
    g_lo = jnp.min(bcol)
    g_hi = jnp.max(bcol)

    def gbody(g, carry):
        m = jnp.max(jnp.where(bcol == g, h, -jnp.inf), axis=0, keepdims=True)
        maxs_sc[pl.ds(g, 1), :] = jnp.maximum(maxs_sc[pl.ds(g, 1), :], m)
        return carry

    lax.fori_loop(g_lo, g_hi + 1, gbody, 0)

    @pl.when(i == NB - 1)
    def _():
        out_ref[...] = jnp.concatenate(
            [sums_sc[...] / jnp.maximum(cnts_sc[...], 1.0), maxs_sc[...]], axis=1
        )


def _tcf(s, stats, g, be, bcol, brow):
    return pl.pallas_call(
        _tcf_body,
        grid=(NB,),
        in_specs=[
            pl.BlockSpec((BR, D), lambda i: (i, 0)),
            pl.BlockSpec((2, D), lambda i: (0, 0)),
            pl.BlockSpec((1, D), lambda i: (0, 0)),
            pl.BlockSpec((1, D), lambda i: (0, 0)),
            pl.BlockSpec((BR, 1), lambda i: (i, 0)),
            pl.BlockSpec((1, 1, BR), lambda i: (i, 0, 0)),
        ],
        out_specs=pl.BlockSpec((G, 2 * D), lambda i: (0, 0)),
        out_shape=jax.ShapeDtypeStruct((G, 2 * D), jnp.float32),
        scratch_shapes=[
            pltpu.VMEM((G, D), jnp.float32),
            pltpu.VMEM((G, 1), jnp.float32),
            pltpu.VMEM((G, D), jnp.float32),
        ],
    )(s, stats, g, be, bcol, brow)


def kernel(x, edge_index, batch, W0, b0, gamma0, beta0, W1, b1, gamma1, beta1,
           W2, b2, gamma2, beta2):
    row = edge_index[0]
    col = edge_index[1]
    row3 = row.reshape(NW, NCH, CH)
    col3 = col.reshape(NW, NCH, CH)
    zeros_nd = jnp.zeros((N, D), jnp.float32)
    zeros_n16 = jnp.zeros((N, 16), jnp.float32)
    ones_ch16 = jnp.ones((CH, 16), jnp.float32)

    sc_degree, sc_agg = _sc_kernels()
    degp = sc_degree(col3, ones_ch16, zeros_n16)
    y, dinvb = _tc0(degp, x, W0)

    Ws = [W1, W2]
    bs = [b0, b1, b2]
    gs = [gamma0, gamma1, gamma2]
    bes = [beta0, beta1, beta2]
    for i in range(3):
        acc = sc_agg(y, row3, col3, zeros_nd)
        if i < 2:
            y = _tcab(acc[0], acc[1], y, dinvb, bs[i].reshape(1, D),
                      gs[i].reshape(1, D), bes[i].reshape(1, D), Ws[i])
        else:
            out = _tcaf(acc[0], acc[1], y, dinvb, bs[i].reshape(1, D),
                        gs[i].reshape(1, D), bes[i].reshape(1, D),
                        batch.reshape(N, 1), batch.reshape(NB, 1, BR))
    return out
